# Initial kernel scaffold; baseline (speedup 1.0000x reference)
import jax
import jax.numpy as jnp
from jax.experimental import pallas as pl


def kernel(x_cat, tables):
    raise NotImplementedError("write your pallas kernel here")


# speedup vs baseline: 1.0363x; 1.0363x over previous
import functools

import jax
import jax.numpy as jnp
from jax import lax
from jax.experimental import pallas as pl
from jax.experimental.pallas import tpu as pltpu
from jax.experimental.pallas import tpu_sc as plsc

N_FIELDS = 26
VOCAB = 100000
HIDDEN = 64
BATCH = 16384

NC = 2
NS = 16
L = 16
NW = NC * NS
ROWS = BATCH * N_FIELDS
RPW = ROWS // NW
CHUNK = 128
NCHUNK = RPW // CHUNK
VPC = CHUNK // L
NBUF = 4


def _body(x_hbm, tab_hbm, out_hbm, idx_v, *rest):
    rows = rest[:NBUF]
    gsem = rest[NBUF:]
    cid = lax.axis_index("c")
    sid = lax.axis_index("s")
    wid = sid * NC + cid
    base_chunk = wid * NCHUNK

    pltpu.sync_copy(x_hbm.at[pl.ds(base_chunk, NCHUNK)], idx_v)

    lane = lax.iota(jnp.int32, L)

    def fix_chunk(c):
        for j in range(VPC):
            r0 = c * CHUNK + j * L
            f = (r0 + lane) % N_FIELDS
            idx_v[c, pl.ds(j * L, L)] = idx_v[c, pl.ds(j * L, L)] + f * VOCAB

    def fire(c, b):
        pltpu.async_copy(tab_hbm.at[idx_v.at[c]], rows[b], gsem[b])

    def wait(b):
        pltpu.make_async_copy(tab_hbm.at[idx_v.at[0]], rows[b], gsem[b]).wait()

    for b in range(NBUF):
        fix_chunk(b)
        fire(b, b)

    def group(g, carry):
        for b in range(NBUF):
            c = g * NBUF + b
            wait(b)
            pltpu.sync_copy(rows[b], out_hbm.at[pl.ds((base_chunk + c) * CHUNK, CHUNK)])
            nxt = c + NBUF

            @pl.when(nxt < NCHUNK)
            def _():
                fix_chunk(nxt)
                fire(nxt, b)
        return carry

    lax.fori_loop(0, NCHUNK // NBUF, group, 0)


def kernel(x_cat, tables):
    x_flat = x_cat.reshape(ROWS // CHUNK, CHUNK)
    tab_flat = tables.reshape(N_FIELDS * VOCAB, HIDDEN)
    mesh = plsc.VectorSubcoreMesh(core_axis_name="c", subcore_axis_name="s")
    run = functools.partial(
        pl.kernel,
        out_type=jax.ShapeDtypeStruct((ROWS, HIDDEN), jnp.float32),
        mesh=mesh,
        compiler_params=pltpu.CompilerParams(use_tc_tiling_on_sc=False),
        scratch_types=[
            pltpu.VMEM((NCHUNK, CHUNK), jnp.int32),
            *[pltpu.VMEM((CHUNK, HIDDEN), jnp.float32) for _ in range(NBUF)],
            *[pltpu.SemaphoreType.DMA for _ in range(NBUF)],
        ],
    )(_body)
    out = run(x_flat, tab_flat)
    return out.reshape(BATCH, N_FIELDS, HIDDEN)


# speedup vs baseline: 3.6410x; 3.5134x over previous
import functools

import jax
import jax.numpy as jnp
from jax import lax
from jax.experimental import pallas as pl
from jax.experimental.pallas import tpu as pltpu
from jax.experimental.pallas import tpu_sc as plsc

N_FIELDS = 26
VOCAB = 100000
HIDDEN = 64
BATCH = 16384

NC = 2
NS = 16
L = 16
NW = NC * NS
NPAIR = N_FIELDS * HIDDEN
PPT = NPAIR // NW
BCHUNK = 8192
UNROLL = 16


def _body(x_hbm, tab_hbm, out_hbm, vocab_v, x_v, res_v):
    cid = lax.axis_index("c")
    sid = lax.axis_index("s")
    wid = sid * NC + cid
    p0 = wid * PPT

    def pair_body(r, prev_f):
        p = p0 + r
        f = p // HIDDEN
        h = p % HIDDEN

        @pl.when(f != prev_f)
        def _():
            pltpu.sync_copy(x_hbm.at[f], x_v)

        pltpu.sync_copy(tab_hbm.at[f, h], vocab_v)

        for c in range(BATCH // BCHUNK):
            def gather_step(j, carry):
                base = j * (UNROLL * L)
                for k in range(UNROLL):
                    o = base + k * L
                    idx = x_v[pl.ds(c * BCHUNK + o, L)]
                    res_v[pl.ds(o, L)] = plsc.load_gather(vocab_v, [idx])
                return carry

            lax.fori_loop(0, BCHUNK // (UNROLL * L), gather_step, 0)
            pltpu.sync_copy(res_v, out_hbm.at[f, h, pl.ds(c * BCHUNK, BCHUNK)])
        return f

    lax.fori_loop(0, PPT, pair_body, -1)


def kernel(x_cat, tables):
    x_t = x_cat.T
    tab_t = tables.transpose(0, 2, 1)
    mesh = plsc.VectorSubcoreMesh(core_axis_name="c", subcore_axis_name="s")
    run = functools.partial(
        pl.kernel,
        out_type=jax.ShapeDtypeStruct((N_FIELDS, HIDDEN, BATCH), jnp.float32),
        mesh=mesh,
        compiler_params=pltpu.CompilerParams(needs_layout_passes=False),
        scratch_types=[
            pltpu.VMEM((VOCAB,), jnp.float32),
            pltpu.VMEM((BATCH,), jnp.int32),
            pltpu.VMEM((BCHUNK,), jnp.float32),
        ],
    )(_body)
    out_t = run(x_t, tab_t)
    return out_t.transpose(2, 0, 1)


# speedup vs baseline: 3.8988x; 1.0708x over previous
import functools

import jax
import jax.numpy as jnp
from jax import lax
from jax.experimental import pallas as pl
from jax.experimental.pallas import tpu as pltpu
from jax.experimental.pallas import tpu_sc as plsc

N_FIELDS = 26
VOCAB = 100000
HIDDEN = 64
BATCH = 16384

NC = 2
NS = 16
L = 16
NW = NC * NS
NPAIR = N_FIELDS * HIDDEN
PPT = NPAIR // NW
BCHUNK = 4096
UNROLL = 16


def _body(x_hbm, tab_hbm, out_hbm, vocab_v, x_v, res0, res1, ssem, osem):
    cid = lax.axis_index("c")
    sid = lax.axis_index("s")
    wid = sid * NC + cid
    p0 = wid * PPT
    res = (res0, res1)

    def pair_body(r, prev_f):
        p = p0 + r
        f = p // HIDDEN
        h = p % HIDDEN

        pltpu.async_copy(tab_hbm.at[f, h], vocab_v, ssem)

        @pl.when(f != prev_f)
        def _():
            pltpu.sync_copy(x_hbm.at[f], x_v)

        pltpu.make_async_copy(tab_hbm.at[f, h], vocab_v, ssem).wait()

        for c in range(BATCH // BCHUNK):
            b = c % 2

            @pl.when((r > 0) | (c > 1))
            def _():
                pltpu.make_async_copy(
                    res[b], out_hbm.at[f, h, pl.ds(0, BCHUNK)], osem
                ).wait()

            def gather_step(j, carry):
                base = j * (UNROLL * L)
                for k in range(UNROLL):
                    o = base + k * L
                    idx = x_v[pl.ds(c * BCHUNK + o, L)]
                    res[b][pl.ds(o, L)] = plsc.load_gather(vocab_v, [idx])
                return carry

            lax.fori_loop(0, BCHUNK // (UNROLL * L), gather_step, 0)
            pltpu.async_copy(
                res[b], out_hbm.at[f, h, pl.ds(c * BCHUNK, BCHUNK)], osem
            )
        return f

    last_f = lax.fori_loop(0, PPT, pair_body, -1)
    for _ in range(2):
        pltpu.make_async_copy(
            res0, out_hbm.at[last_f, 0, pl.ds(0, BCHUNK)], osem
        ).wait()


def kernel(x_cat, tables):
    x_t = x_cat.T
    tab_t = tables.transpose(0, 2, 1)
    mesh = plsc.VectorSubcoreMesh(core_axis_name="c", subcore_axis_name="s")
    run = functools.partial(
        pl.kernel,
        out_type=jax.ShapeDtypeStruct((N_FIELDS, HIDDEN, BATCH), jnp.float32),
        mesh=mesh,
        compiler_params=pltpu.CompilerParams(needs_layout_passes=False),
        scratch_types=[
            pltpu.VMEM((VOCAB,), jnp.float32),
            pltpu.VMEM((BATCH,), jnp.int32),
            pltpu.VMEM((BCHUNK,), jnp.float32),
            pltpu.VMEM((BCHUNK,), jnp.float32),
            pltpu.SemaphoreType.DMA,
            pltpu.SemaphoreType.DMA,
        ],
    )(_body)
    out_t = run(x_t, tab_t)
    return out_t.transpose(2, 0, 1)


# speedup vs baseline: 3.9051x; 1.0016x over previous
import functools

import jax
import jax.numpy as jnp
from jax import lax
from jax.experimental import pallas as pl
from jax.experimental.pallas import tpu as pltpu
from jax.experimental.pallas import tpu_sc as plsc

N_FIELDS = 26
VOCAB = 100000
HIDDEN = 64
BATCH = 16384

NC = 2
NS = 16
L = 16
NW = NC * NS
NPAIR = N_FIELDS * HIDDEN
PPT = NPAIR // NW
BCHUNK = 4096
UNROLL = 32


def _body(x_hbm, tab_hbm, out_hbm, vocab_v, x_v, res0, res1, ssem, osem):
    cid = lax.axis_index("c")
    sid = lax.axis_index("s")
    wid = sid * NC + cid
    p0 = wid * PPT
    res = (res0, res1)

    def pair_body(r, prev_f):
        p = p0 + r
        f = p // HIDDEN
        h = p % HIDDEN

        pltpu.async_copy(tab_hbm.at[f, h], vocab_v, ssem)

        @pl.when(f != prev_f)
        def _():
            pltpu.sync_copy(x_hbm.at[f], x_v)

        pltpu.make_async_copy(tab_hbm.at[f, h], vocab_v, ssem).wait()

        for c in range(BATCH // BCHUNK):
            b = c % 2

            @pl.when((r > 0) | (c > 1))
            def _():
                pltpu.make_async_copy(
                    res[b], out_hbm.at[f, h, pl.ds(0, BCHUNK)], osem
                ).wait()

            def gather_step(j, carry):
                base = j * (UNROLL * L)
                for k in range(UNROLL):
                    o = base + k * L
                    idx = x_v[pl.ds(c * BCHUNK + o, L)]
                    res[b][pl.ds(o, L)] = plsc.load_gather(vocab_v, [idx])
                return carry

            lax.fori_loop(0, BCHUNK // (UNROLL * L), gather_step, 0)
            pltpu.async_copy(
                res[b], out_hbm.at[f, h, pl.ds(c * BCHUNK, BCHUNK)], osem
            )
        return f

    last_f = lax.fori_loop(0, PPT, pair_body, -1)
    for _ in range(2):
        pltpu.make_async_copy(
            res0, out_hbm.at[last_f, 0, pl.ds(0, BCHUNK)], osem
        ).wait()


def kernel(x_cat, tables):
    x_t = x_cat.T
    tab_t = tables.transpose(0, 2, 1)
    mesh = plsc.VectorSubcoreMesh(core_axis_name="c", subcore_axis_name="s")
    run = functools.partial(
        pl.kernel,
        out_type=jax.ShapeDtypeStruct((N_FIELDS, HIDDEN, BATCH), jnp.float32),
        mesh=mesh,
        compiler_params=pltpu.CompilerParams(needs_layout_passes=False),
        scratch_types=[
            pltpu.VMEM((VOCAB,), jnp.float32),
            pltpu.VMEM((BATCH,), jnp.int32),
            pltpu.VMEM((BCHUNK,), jnp.float32),
            pltpu.VMEM((BCHUNK,), jnp.float32),
            pltpu.SemaphoreType.DMA,
            pltpu.SemaphoreType.DMA,
        ],
    )(_body)
    out_t = run(x_t, tab_t)
    return out_t.transpose(2, 0, 1)


# speedup vs baseline: 5.5936x; 1.4324x over previous
import functools

import jax
import jax.numpy as jnp
from jax import lax
from jax.experimental import pallas as pl
from jax.experimental.pallas import tpu as pltpu
from jax.experimental.pallas import tpu_sc as plsc

N_FIELDS = 26
VOCAB = 100000
HIDDEN = 64
BATCH = 16384

NC = 2
NS = 16
L = 16
NW = NC * NS
NPAIR = N_FIELDS * HIDDEN
PPT = NPAIR // NW
BCHUNK = 4096
UNROLL = 32


def _body(x_hbm, tab_hbm, out_hbm, vocab_v, x_v, res0, res1, ssem, osem):
    cid = lax.axis_index("c")
    sid = lax.axis_index("s")
    wid = sid * NC + cid
    p0 = wid * PPT
    res = (res0, res1)

    def pair_body(r, prev_f):
        p = p0 + r
        f = p // HIDDEN
        h = p % HIDDEN

        pltpu.async_copy(tab_hbm.at[f, h], vocab_v, ssem)

        @pl.when(f != prev_f)
        def _():
            pltpu.sync_copy(x_hbm.at[f], x_v)

        pltpu.make_async_copy(tab_hbm.at[f, h], vocab_v, ssem).wait()

        for c in range(BATCH // BCHUNK):
            b = c % 2

            @pl.when((r > 0) | (c > 1))
            def _():
                pltpu.make_async_copy(
                    res[b], out_hbm.at[f, h, pl.ds(0, BCHUNK)], osem
                ).wait()

            @plsc.parallel_loop(0, BCHUNK, step=L, unroll=UNROLL)
            def _(o):
                idx = x_v[pl.ds(c * BCHUNK + o, L)]
                res[b][pl.ds(o, L)] = plsc.load_gather(vocab_v, [idx])
            pltpu.async_copy(
                res[b], out_hbm.at[f, h, pl.ds(c * BCHUNK, BCHUNK)], osem
            )
        return f

    last_f = lax.fori_loop(0, PPT, pair_body, -1)
    for _ in range(2):
        pltpu.make_async_copy(
            res0, out_hbm.at[last_f, 0, pl.ds(0, BCHUNK)], osem
        ).wait()


def kernel(x_cat, tables):
    x_t = x_cat.T
    tab_t = tables.transpose(0, 2, 1)
    mesh = plsc.VectorSubcoreMesh(core_axis_name="c", subcore_axis_name="s")
    run = functools.partial(
        pl.kernel,
        out_type=jax.ShapeDtypeStruct((N_FIELDS, HIDDEN, BATCH), jnp.float32),
        mesh=mesh,
        compiler_params=pltpu.CompilerParams(needs_layout_passes=False),
        scratch_types=[
            pltpu.VMEM((VOCAB,), jnp.float32),
            pltpu.VMEM((BATCH,), jnp.int32),
            pltpu.VMEM((BCHUNK,), jnp.float32),
            pltpu.VMEM((BCHUNK,), jnp.float32),
            pltpu.SemaphoreType.DMA,
            pltpu.SemaphoreType.DMA,
        ],
    )(_body)
    out_t = run(x_t, tab_t)
    return out_t.transpose(2, 0, 1)
